# Initial kernel scaffold; baseline (speedup 1.0000x reference)
import jax
import jax.numpy as jnp
from jax.experimental import pallas as pl


def kernel(support_features, query_features, support_labels):
    raise NotImplementedError("write your pallas kernel here")


# speedup vs baseline: 13.6805x; 13.6805x over previous
import jax
import jax.numpy as jnp
from jax.experimental import pallas as pl
from jax.experimental.pallas import tpu as pltpu

_N_S = 4096
_N_Q = 2048
_D = 512
_C = 2
_NEWTON_ITERS = 14


def _proto_kernel(x_ref, q_ref, lab_ref, out_ref):
    X = x_ref[...]
    labs = lab_ref[...]
    mask1 = (labs == 1).astype(jnp.float32)
    Xm = X * mask1

    n1 = jnp.sum(mask1)
    n0 = _N_S - n1

    dnums = (((0,), (0,)), ((), ()))
    G_tot = jax.lax.dot_general(X, X, dnums,
                                preferred_element_type=jnp.float32)
    G_1 = jax.lax.dot_general(Xm, X, dnums,
                              preferred_element_type=jnp.float32)
    G_0 = G_tot - G_1

    s_tot = jnp.sum(X, axis=0, keepdims=True)
    s_1 = jnp.sum(Xm, axis=0, keepdims=True)
    s_0 = s_tot - s_1

    m_all = s_tot / _N_S
    task_cov = (G_tot - _N_S * (m_all.T * m_all)) / (_N_S - 1.0)

    row = jax.lax.broadcasted_iota(jnp.int32, (_D, _D), 0)
    col = jax.lax.broadcasted_iota(jnp.int32, (_D, _D), 1)
    eye = (row == col).astype(jnp.float32)

    Q = q_ref[...]

    logits = []
    for c, (G_c, s_c, n_c) in enumerate(((G_0, s_0, n0), (G_1, s_1, n1))):
        m_c = s_c / n_c
        cov_c = (G_c - n_c * (m_c.T * m_c)) / (n_c - 1.0)
        lam = jnp.minimum(n_c / (n_c + 1.0), 0.1)
        S = lam * cov_c + (1.0 - lam) * task_cov + 0.1 * eye

        gersh = jnp.max(jnp.sum(jnp.abs(S), axis=1))
        c0 = 2.0 / (gersh + 0.1)

        def newton_body(_, P):
            SP = jnp.dot(S, P, preferred_element_type=jnp.float32)
            return jnp.dot(P, 2.0 * eye - SP,
                           preferred_element_type=jnp.float32)

        P = jax.lax.fori_loop(0, _NEWTON_ITERS, newton_body, c0 * eye)

        A = jnp.dot(Q, P, preferred_element_type=jnp.float32)
        quad = jnp.sum(A * Q, axis=1, keepdims=True)
        cross = jnp.dot(A, m_c.T, preferred_element_type=jnp.float32)
        mP = jnp.dot(m_c, P, preferred_element_type=jnp.float32)
        const = jnp.sum(mP * m_c)
        logits.append(-(quad - 2.0 * cross + const))

    out_ref[...] = jnp.concatenate(logits, axis=1)


def kernel(support_features, query_features, support_labels):
    labs2d = support_labels.reshape(_N_S, 1).astype(jnp.int32)
    return pl.pallas_call(
        _proto_kernel,
        out_shape=jax.ShapeDtypeStruct((_N_Q, _C), jnp.float32),
        compiler_params=pltpu.CompilerParams(
            vmem_limit_bytes=100 * 1024 * 1024,
        ),
    )(support_features, query_features, labs2d)


# speedup vs baseline: 16.0304x; 1.1718x over previous
import jax
import jax.numpy as jnp
from jax.experimental import pallas as pl
from jax.experimental.pallas import tpu as pltpu

_N_S = 4096
_N_Q = 2048
_D = 512
_C = 2
_NEWTON_ITERS_BF16 = 8
_NEWTON_ITERS_F32 = 3


def _proto_kernel(x_ref, q_ref, lab_ref, out_ref):
    X = x_ref[...]
    labs = lab_ref[...]
    mask1 = (labs == 1).astype(jnp.float32)
    Xm = X * mask1

    n1 = jnp.sum(mask1)
    n0 = _N_S - n1

    dnums = (((0,), (0,)), ((), ()))
    G_tot = jax.lax.dot_general(X, X, dnums,
                                preferred_element_type=jnp.float32)
    G_1 = jax.lax.dot_general(Xm, X, dnums,
                              preferred_element_type=jnp.float32)
    G_0 = G_tot - G_1

    s_tot = jnp.sum(X, axis=0, keepdims=True)
    s_1 = jnp.sum(Xm, axis=0, keepdims=True)
    s_0 = s_tot - s_1

    m_all = s_tot / _N_S
    task_cov = (G_tot - _N_S * (m_all.T * m_all)) / (_N_S - 1.0)

    row = jax.lax.broadcasted_iota(jnp.int32, (_D, _D), 0)
    col = jax.lax.broadcasted_iota(jnp.int32, (_D, _D), 1)
    eye = (row == col).astype(jnp.float32)

    Q = q_ref[...]

    logits = []
    for c, (G_c, s_c, n_c) in enumerate(((G_0, s_0, n0), (G_1, s_1, n1))):
        m_c = s_c / n_c
        cov_c = (G_c - n_c * (m_c.T * m_c)) / (n_c - 1.0)
        lam = jnp.minimum(n_c / (n_c + 1.0), 0.1)
        S = lam * cov_c + (1.0 - lam) * task_cov + 0.1 * eye

        gersh = jnp.max(jnp.sum(jnp.abs(S), axis=1))
        c0 = 2.0 / (gersh + 0.1)

        S_bf = S.astype(jnp.bfloat16)

        def newton_bf16(_, P):
            SP = jnp.dot(S_bf, P, preferred_element_type=jnp.float32)
            T = (2.0 * eye - SP).astype(jnp.bfloat16)
            return jnp.dot(P, T,
                           preferred_element_type=jnp.float32
                           ).astype(jnp.bfloat16)

        P = jax.lax.fori_loop(0, _NEWTON_ITERS_BF16, newton_bf16,
                              (c0 * eye).astype(jnp.bfloat16))
        P = P.astype(jnp.float32)

        def newton_f32(_, P):
            SP = jnp.dot(S, P, preferred_element_type=jnp.float32)
            return jnp.dot(P, 2.0 * eye - SP,
                           preferred_element_type=jnp.float32)

        P = jax.lax.fori_loop(0, _NEWTON_ITERS_F32, newton_f32, P)

        A = jnp.dot(Q, P, preferred_element_type=jnp.float32)
        quad = jnp.sum(A * Q, axis=1, keepdims=True)
        cross = jnp.dot(A, m_c.T, preferred_element_type=jnp.float32)
        mP = jnp.dot(m_c, P, preferred_element_type=jnp.float32)
        const = jnp.sum(mP * m_c)
        logits.append(-(quad - 2.0 * cross + const))

    out_ref[...] = jnp.concatenate(logits, axis=1)


def kernel(support_features, query_features, support_labels):
    labs2d = support_labels.reshape(_N_S, 1).astype(jnp.int32)
    return pl.pallas_call(
        _proto_kernel,
        out_shape=jax.ShapeDtypeStruct((_N_Q, _C), jnp.float32),
        compiler_params=pltpu.CompilerParams(
            vmem_limit_bytes=100 * 1024 * 1024,
        ),
    )(support_features, query_features, labs2d)


# speedup vs baseline: 17.8051x; 1.1107x over previous
import jax
import jax.numpy as jnp
from jax.experimental import pallas as pl
from jax.experimental.pallas import tpu as pltpu

_N_S = 4096
_N_Q = 2048
_D = 512
_C = 2
_NEWTON_ITERS_BF16 = 7
_NEWTON_ITERS_F32 = 2


def _proto_kernel(x_ref, q_ref, lab_ref, out_ref):
    X = x_ref[...]
    labs = lab_ref[...]
    mask1 = (labs == 1).astype(jnp.float32)
    Xm = X * mask1

    n1 = jnp.sum(mask1)
    n0 = _N_S - n1

    dnums = (((0,), (0,)), ((), ()))
    G_tot = jax.lax.dot_general(X, X, dnums,
                                preferred_element_type=jnp.float32)
    G_1 = jax.lax.dot_general(Xm, X, dnums,
                              preferred_element_type=jnp.float32)
    G_0 = G_tot - G_1

    s_tot = jnp.sum(X, axis=0, keepdims=True)
    s_1 = jnp.sum(Xm, axis=0, keepdims=True)
    s_0 = s_tot - s_1

    m_all = s_tot / _N_S
    task_cov = (G_tot - _N_S * (m_all.T * m_all)) / (_N_S - 1.0)

    row = jax.lax.broadcasted_iota(jnp.int32, (_D, _D), 0)
    col = jax.lax.broadcasted_iota(jnp.int32, (_D, _D), 1)
    eye = (row == col).astype(jnp.float32)

    precisions = []
    means = []
    for c, (G_c, s_c, n_c) in enumerate(((G_0, s_0, n0), (G_1, s_1, n1))):
        m_c = s_c / n_c
        cov_c = (G_c - n_c * (m_c.T * m_c)) / (n_c - 1.0)
        lam = jnp.minimum(n_c / (n_c + 1.0), 0.1)
        S = lam * cov_c + (1.0 - lam) * task_cov + 0.1 * eye

        gersh = jnp.max(jnp.sum(jnp.abs(S), axis=1))
        c0 = 2.0 / (gersh + 0.1)

        S_bf = S.astype(jnp.bfloat16)

        def newton_bf16(_, P):
            SP = jnp.dot(S_bf, P, preferred_element_type=jnp.float32)
            T = (2.0 * eye - SP).astype(jnp.bfloat16)
            return jnp.dot(P, T,
                           preferred_element_type=jnp.float32
                           ).astype(jnp.bfloat16)

        P = jax.lax.fori_loop(0, _NEWTON_ITERS_BF16, newton_bf16,
                              (c0 * eye).astype(jnp.bfloat16))
        P = P.astype(jnp.float32)

        def newton_f32(_, P):
            SP = jnp.dot(S, P, preferred_element_type=jnp.float32)
            return jnp.dot(P, 2.0 * eye - SP,
                           preferred_element_type=jnp.float32)

        P = jax.lax.fori_loop(0, _NEWTON_ITERS_F32, newton_f32, P)
        precisions.append(P)
        means.append(m_c)

    Q = q_ref[...]
    Q_bf = Q.astype(jnp.bfloat16)
    Pcat = jnp.concatenate(precisions, axis=1).astype(jnp.bfloat16)
    A = jnp.dot(Q_bf, Pcat, preferred_element_type=jnp.float32)

    logits = []
    for c in range(_C):
        A_c = A[:, c * _D:(c + 1) * _D]
        m_c = means[c]
        P_c = precisions[c]
        quad = jnp.sum(A_c * Q, axis=1, keepdims=True)
        cross = jnp.dot(A_c, m_c.T, preferred_element_type=jnp.float32)
        mP = jnp.dot(m_c, P_c, preferred_element_type=jnp.float32)
        const = jnp.sum(mP * m_c)
        logits.append(-(quad - 2.0 * cross + const))

    out_ref[...] = jnp.concatenate(logits, axis=1)


def kernel(support_features, query_features, support_labels):
    labs2d = support_labels.reshape(_N_S, 1).astype(jnp.int32)
    return pl.pallas_call(
        _proto_kernel,
        out_shape=jax.ShapeDtypeStruct((_N_Q, _C), jnp.float32),
        compiler_params=pltpu.CompilerParams(
            vmem_limit_bytes=100 * 1024 * 1024,
        ),
    )(support_features, query_features, labs2d)


# speedup vs baseline: 19.4173x; 1.0905x over previous
import jax
import jax.numpy as jnp
from jax.experimental import pallas as pl
from jax.experimental.pallas import tpu as pltpu

_N_S = 4096
_N_Q = 2048
_D = 512
_C = 2
_NEWTON_ITERS_BF16 = 7
_NEWTON_ITERS_WARM = 4
_NEWTON_ITERS_F32 = 2


def _proto_kernel(x_ref, q_ref, lab_ref, out_ref):
    X = x_ref[...]
    labs = lab_ref[...]
    mask1 = (labs == 1).astype(jnp.float32)
    Xm = X * mask1

    n1 = jnp.sum(mask1)
    n0 = _N_S - n1

    dnums = (((0,), (0,)), ((), ()))
    G_tot = jax.lax.dot_general(X, X, dnums,
                                preferred_element_type=jnp.float32)
    X_bf = X.astype(jnp.bfloat16)
    Xm_bf = Xm.astype(jnp.bfloat16)
    G_1 = jax.lax.dot_general(Xm_bf, X_bf, dnums,
                              preferred_element_type=jnp.float32)
    G_0 = G_tot - G_1

    s_tot = jnp.sum(X, axis=0, keepdims=True)
    s_1 = jnp.sum(Xm, axis=0, keepdims=True)
    s_0 = s_tot - s_1

    m_all = s_tot / _N_S
    task_cov = (G_tot - _N_S * (m_all.T * m_all)) / (_N_S - 1.0)

    row = jax.lax.broadcasted_iota(jnp.int32, (_D, _D), 0)
    col = jax.lax.broadcasted_iota(jnp.int32, (_D, _D), 1)
    eye = (row == col).astype(jnp.float32)

    precisions = []
    means = []
    for c, (G_c, s_c, n_c) in enumerate(((G_0, s_0, n0), (G_1, s_1, n1))):
        m_c = s_c / n_c
        cov_c = (G_c - n_c * (m_c.T * m_c)) / (n_c - 1.0)
        lam = jnp.minimum(n_c / (n_c + 1.0), 0.1)
        S = lam * cov_c + (1.0 - lam) * task_cov + 0.1 * eye

        gersh = jnp.max(jnp.sum(jnp.abs(S), axis=1))
        c0 = 2.0 / (gersh + 0.1)

        S_bf = S.astype(jnp.bfloat16)

        def newton_bf16(_, P):
            SP = jnp.dot(S_bf, P, preferred_element_type=jnp.float32)
            T = (2.0 * eye - SP).astype(jnp.bfloat16)
            return jnp.dot(P, T,
                           preferred_element_type=jnp.float32
                           ).astype(jnp.bfloat16)

        if c == 0:
            P = jax.lax.fori_loop(0, _NEWTON_ITERS_BF16, newton_bf16,
                                  (c0 * eye).astype(jnp.bfloat16))
        else:
            P = jax.lax.fori_loop(0, _NEWTON_ITERS_WARM, newton_bf16,
                                  precisions[0].astype(jnp.bfloat16))
        P = P.astype(jnp.float32)

        def newton_f32(_, P):
            SP = jnp.dot(S, P, preferred_element_type=jnp.float32)
            return jnp.dot(P, 2.0 * eye - SP,
                           preferred_element_type=jnp.float32)

        P = jax.lax.fori_loop(0, _NEWTON_ITERS_F32, newton_f32, P)
        precisions.append(P)
        means.append(m_c)

    Q = q_ref[...]
    Q_bf = Q.astype(jnp.bfloat16)
    Pcat = jnp.concatenate(precisions, axis=1).astype(jnp.bfloat16)
    A = jnp.dot(Q_bf, Pcat, preferred_element_type=jnp.float32)

    logits = []
    for c in range(_C):
        A_c = A[:, c * _D:(c + 1) * _D]
        m_c = means[c]
        P_c = precisions[c]
        quad = jnp.sum(A_c * Q, axis=1, keepdims=True)
        cross = jnp.dot(A_c, m_c.T, preferred_element_type=jnp.float32)
        mP = jnp.dot(m_c, P_c, preferred_element_type=jnp.float32)
        const = jnp.sum(mP * m_c)
        logits.append(-(quad - 2.0 * cross + const))

    out_ref[...] = jnp.concatenate(logits, axis=1)


def kernel(support_features, query_features, support_labels):
    labs2d = support_labels.reshape(_N_S, 1).astype(jnp.int32)
    return pl.pallas_call(
        _proto_kernel,
        out_shape=jax.ShapeDtypeStruct((_N_Q, _C), jnp.float32),
        compiler_params=pltpu.CompilerParams(
            vmem_limit_bytes=100 * 1024 * 1024,
        ),
    )(support_features, query_features, labs2d)


# speedup vs baseline: 19.4406x; 1.0012x over previous
import jax
import jax.numpy as jnp
from jax.experimental import pallas as pl
from jax.experimental.pallas import tpu as pltpu

_N_S = 4096
_N_Q = 2048
_D = 512
_C = 2
_NEWTON_ITERS_BF16 = 7
_NEWTON_ITERS_WARM = 4
_NEWTON_ITERS_F32 = 2


def _proto_kernel(x_ref, q_ref, lab_ref, out_ref):
    X = x_ref[...]
    labs = lab_ref[...]
    mask1 = (labs == 1).astype(jnp.float32)
    Xm = X * mask1

    n1 = jnp.sum(mask1)
    n0 = _N_S - n1

    dnums = (((0,), (0,)), ((), ()))
    G_tot = jax.lax.dot_general(X, X, dnums,
                                preferred_element_type=jnp.float32)
    X_bf = X.astype(jnp.bfloat16)
    Xm_bf = Xm.astype(jnp.bfloat16)
    G_1 = jax.lax.dot_general(Xm_bf, X_bf, dnums,
                              preferred_element_type=jnp.float32)
    G_0 = G_tot - G_1

    s_tot = jnp.sum(X, axis=0, keepdims=True)
    s_1 = jnp.sum(Xm, axis=0, keepdims=True)
    s_0 = s_tot - s_1

    m_all = s_tot / _N_S
    task_cov = (G_tot - _N_S * (m_all.T * m_all)) / (_N_S - 1.0)

    row = jax.lax.broadcasted_iota(jnp.int32, (_D, _D), 0)
    col = jax.lax.broadcasted_iota(jnp.int32, (_D, _D), 1)
    eye = (row == col).astype(jnp.float32)

    precisions = []
    means = []
    for c, (G_c, s_c, n_c) in enumerate(((G_0, s_0, n0), (G_1, s_1, n1))):
        m_c = s_c / n_c
        cov_c = (G_c - n_c * (m_c.T * m_c)) / (n_c - 1.0)
        lam = jnp.minimum(n_c / (n_c + 1.0), 0.1)
        S = lam * cov_c + (1.0 - lam) * task_cov + 0.1 * eye

        gersh = jnp.max(jnp.sum(jnp.abs(S), axis=1))
        c0 = 2.0 / (gersh + 0.1)

        S_bf = S.astype(jnp.bfloat16)

        def newton_bf16(_, P):
            SP = jnp.dot(S_bf, P, preferred_element_type=jnp.float32)
            T = (2.0 * eye - SP).astype(jnp.bfloat16)
            return jnp.dot(P, T,
                           preferred_element_type=jnp.float32
                           ).astype(jnp.bfloat16)

        if c == 0:
            P = jax.lax.fori_loop(0, _NEWTON_ITERS_BF16, newton_bf16,
                                  (c0 * eye).astype(jnp.bfloat16))
        else:
            P = jax.lax.fori_loop(0, _NEWTON_ITERS_WARM, newton_bf16,
                                  precisions[0].astype(jnp.bfloat16))
        P = P.astype(jnp.float32)

        def newton_polish(_, P):
            SP = jnp.dot(S, P, preferred_element_type=jnp.float32)
            E = (eye - SP).astype(jnp.bfloat16)
            dP = jnp.dot(P.astype(jnp.bfloat16), E,
                         preferred_element_type=jnp.float32)
            return P + dP

        P = jax.lax.fori_loop(0, _NEWTON_ITERS_F32, newton_polish, P)
        precisions.append(P)
        means.append(m_c)

    Q = q_ref[...]
    Q_bf = Q.astype(jnp.bfloat16)
    Pcat = jnp.concatenate(precisions, axis=1).astype(jnp.bfloat16)
    A = jnp.dot(Q_bf, Pcat, preferred_element_type=jnp.float32)

    logits = []
    for c in range(_C):
        A_c = A[:, c * _D:(c + 1) * _D]
        m_c = means[c]
        P_c = precisions[c]
        quad = jnp.sum(A_c * Q, axis=1, keepdims=True)
        cross = jnp.dot(A_c, m_c.T, preferred_element_type=jnp.float32)
        mP = jnp.dot(m_c, P_c, preferred_element_type=jnp.float32)
        const = jnp.sum(mP * m_c)
        logits.append(-(quad - 2.0 * cross + const))

    out_ref[...] = jnp.concatenate(logits, axis=1)


def kernel(support_features, query_features, support_labels):
    labs2d = support_labels.reshape(_N_S, 1).astype(jnp.int32)
    return pl.pallas_call(
        _proto_kernel,
        out_shape=jax.ShapeDtypeStruct((_N_Q, _C), jnp.float32),
        compiler_params=pltpu.CompilerParams(
            vmem_limit_bytes=100 * 1024 * 1024,
        ),
    )(support_features, query_features, labs2d)


# speedup vs baseline: 19.7150x; 1.0141x over previous
import jax
import jax.numpy as jnp
from jax.experimental import pallas as pl
from jax.experimental.pallas import tpu as pltpu

_N_S = 4096
_N_Q = 2048
_D = 512
_C = 2
_NEWTON_ITERS_BF16 = 6
_NEWTON_ITERS_WARM = 3
_NEWTON_ITERS_F32 = 2


def _proto_kernel(x_ref, q_ref, lab_ref, out_ref):
    X = x_ref[...]
    labs = lab_ref[...]
    mask1 = (labs == 1).astype(jnp.float32)

    n1 = jnp.sum(mask1)
    n0 = _N_S - n1

    dnums = (((0,), (0,)), ((), ()))

    X_hi = X.astype(jnp.bfloat16)
    X_lo = (X - X_hi.astype(jnp.float32)).astype(jnp.bfloat16)
    G_hh = jax.lax.dot_general(X_hi, X_hi, dnums,
                               preferred_element_type=jnp.float32)
    C = jax.lax.dot_general(X_hi, X_lo, dnums,
                            preferred_element_type=jnp.float32)
    G_tot = G_hh + C + C.T

    Xm_hi = X_hi * mask1.astype(jnp.bfloat16)
    G_1 = jax.lax.dot_general(Xm_hi, Xm_hi, dnums,
                              preferred_element_type=jnp.float32)
    G_0 = G_tot - G_1

    ones_col = jnp.ones((_N_S, 1), dtype=jnp.float32)
    s_tot = jax.lax.dot_general(ones_col, X, dnums,
                                preferred_element_type=jnp.float32)
    s_1 = jax.lax.dot_general(mask1, X, dnums,
                              preferred_element_type=jnp.float32)
    s_0 = s_tot - s_1

    m_all = s_tot / _N_S
    task_cov = (G_tot - _N_S * (m_all.T * m_all)) / (_N_S - 1.0)

    row = jax.lax.broadcasted_iota(jnp.int32, (_D, _D), 0)
    col = jax.lax.broadcasted_iota(jnp.int32, (_D, _D), 1)
    eye = (row == col).astype(jnp.float32)

    precisions = []
    means = []
    for c, (G_c, s_c, n_c) in enumerate(((G_0, s_0, n0), (G_1, s_1, n1))):
        m_c = s_c / n_c
        cov_c = (G_c - n_c * (m_c.T * m_c)) / (n_c - 1.0)
        lam = jnp.minimum(n_c / (n_c + 1.0), 0.1)
        S = lam * cov_c + (1.0 - lam) * task_cov + 0.1 * eye

        gersh = jnp.max(jnp.sum(jnp.abs(S), axis=1))
        c0 = 2.0 / (gersh + 0.1)

        S_bf = S.astype(jnp.bfloat16)

        def newton_bf16(_, P):
            SP = jnp.dot(S_bf, P, preferred_element_type=jnp.float32)
            T = (2.0 * eye - SP).astype(jnp.bfloat16)
            return jnp.dot(P, T,
                           preferred_element_type=jnp.float32
                           ).astype(jnp.bfloat16)

        if c == 0:
            P = jax.lax.fori_loop(0, _NEWTON_ITERS_BF16, newton_bf16,
                                  (c0 * eye).astype(jnp.bfloat16))
        else:
            P = jax.lax.fori_loop(0, _NEWTON_ITERS_WARM, newton_bf16,
                                  precisions[0].astype(jnp.bfloat16))
        P = P.astype(jnp.float32)

        def newton_polish(_, P):
            SP = jnp.dot(S, P, preferred_element_type=jnp.float32)
            E = (eye - SP).astype(jnp.bfloat16)
            dP = jnp.dot(P.astype(jnp.bfloat16), E,
                         preferred_element_type=jnp.float32)
            return P + dP

        P = jax.lax.fori_loop(0, _NEWTON_ITERS_F32, newton_polish, P)
        precisions.append(P)
        means.append(m_c)

    Q = q_ref[...]
    Q_bf = Q.astype(jnp.bfloat16)
    Pcat = jnp.concatenate(precisions, axis=1).astype(jnp.bfloat16)
    A = jnp.dot(Q_bf, Pcat, preferred_element_type=jnp.float32)

    logits = []
    for c in range(_C):
        A_c = A[:, c * _D:(c + 1) * _D]
        m_c = means[c]
        P_c = precisions[c]
        quad = jnp.sum(A_c * Q, axis=1, keepdims=True)
        cross = jnp.dot(A_c, m_c.T, preferred_element_type=jnp.float32)
        mP = jnp.dot(m_c, P_c, preferred_element_type=jnp.float32)
        const = jnp.sum(mP * m_c)
        logits.append(-(quad - 2.0 * cross + const))

    out_ref[...] = jnp.concatenate(logits, axis=1)


def kernel(support_features, query_features, support_labels):
    labs2d = support_labels.reshape(_N_S, 1).astype(jnp.int32)
    return pl.pallas_call(
        _proto_kernel,
        out_shape=jax.ShapeDtypeStruct((_N_Q, _C), jnp.float32),
        compiler_params=pltpu.CompilerParams(
            vmem_limit_bytes=100 * 1024 * 1024,
        ),
    )(support_features, query_features, labs2d)


# speedup vs baseline: 20.5333x; 1.0415x over previous
import jax
import jax.numpy as jnp
from jax.experimental import pallas as pl
from jax.experimental.pallas import tpu as pltpu

_N_S = 4096
_N_Q = 2048
_D = 512
_C = 2
_NEWTON_ITERS_BF16 = 6
_NEWTON_ITERS_WARM = 3
_NEWTON_ITERS_F32 = 2


def _proto_kernel(x_ref, q_ref, lab_ref, out_ref):
    X = x_ref[...]
    labs = lab_ref[...]
    mask1 = (labs == 1).astype(jnp.float32)

    n1 = jnp.sum(mask1)
    n0 = _N_S - n1

    dnums = (((0,), (0,)), ((), ()))

    row = jax.lax.broadcasted_iota(jnp.int32, (_D, _D), 0)
    col = jax.lax.broadcasted_iota(jnp.int32, (_D, _D), 1)
    eye = (row == col).astype(jnp.float32)

    X_hi = X.astype(jnp.bfloat16)
    G_tot = jax.lax.dot_general(X_hi, X_hi, dnums,
                                preferred_element_type=jnp.float32)

    Xm_hi = X_hi * mask1.astype(jnp.bfloat16)
    G_1 = jax.lax.dot_general(Xm_hi, Xm_hi, dnums,
                              preferred_element_type=jnp.float32)
    G_0 = G_tot - G_1

    s_tot = jnp.sum(X, axis=0, keepdims=True)
    s_1 = jnp.sum(X * mask1, axis=0, keepdims=True)
    s_0 = s_tot - s_1

    m_all = s_tot / _N_S
    task_cov = (G_tot - _N_S * (m_all.T * m_all)) / (_N_S - 1.0)

    row = jax.lax.broadcasted_iota(jnp.int32, (_D, _D), 0)
    col = jax.lax.broadcasted_iota(jnp.int32, (_D, _D), 1)
    eye = (row == col).astype(jnp.float32)

    precisions = []
    means = []
    for c, (G_c, s_c, n_c) in enumerate(((G_0, s_0, n0), (G_1, s_1, n1))):
        m_c = s_c / n_c
        cov_c = (G_c - n_c * (m_c.T * m_c)) / (n_c - 1.0)
        lam = jnp.minimum(n_c / (n_c + 1.0), 0.1)
        S = lam * cov_c + (1.0 - lam) * task_cov + 0.1 * eye

        gersh = jnp.max(jnp.sum(jnp.abs(S), axis=1))
        c0 = 2.0 / (gersh + 0.1)

        S_bf = S.astype(jnp.bfloat16)

        def newton_bf16(_, P):
            SP = jnp.dot(S_bf, P, preferred_element_type=jnp.float32)
            T = (2.0 * eye - SP).astype(jnp.bfloat16)
            return jnp.dot(P, T,
                           preferred_element_type=jnp.float32
                           ).astype(jnp.bfloat16)

        if c == 0:
            P = jax.lax.fori_loop(0, _NEWTON_ITERS_BF16, newton_bf16,
                                  (c0 * eye).astype(jnp.bfloat16))
        else:
            P = jax.lax.fori_loop(0, _NEWTON_ITERS_WARM, newton_bf16,
                                  precisions[0].astype(jnp.bfloat16))
        P = P.astype(jnp.float32)

        def newton_polish(_, P):
            SP = jnp.dot(S, P, preferred_element_type=jnp.float32)
            E = (eye - SP).astype(jnp.bfloat16)
            dP = jnp.dot(P.astype(jnp.bfloat16), E,
                         preferred_element_type=jnp.float32)
            return P + dP

        P = jax.lax.fori_loop(0, _NEWTON_ITERS_F32, newton_polish, P)
        precisions.append(P)
        means.append(m_c)

    Q = q_ref[...]
    Q_bf = Q.astype(jnp.bfloat16)
    Pcat = jnp.concatenate(precisions, axis=1).astype(jnp.bfloat16)
    A = jnp.dot(Q_bf, Pcat, preferred_element_type=jnp.float32)

    logits = []
    for c in range(_C):
        A_c = A[:, c * _D:(c + 1) * _D]
        m_c = means[c]
        P_c = precisions[c]
        quad = jnp.sum(A_c * Q, axis=1, keepdims=True)
        cross = jnp.dot(A_c, m_c.T, preferred_element_type=jnp.float32)
        mP = jnp.dot(m_c, P_c, preferred_element_type=jnp.float32)
        const = jnp.sum(mP * m_c)
        logits.append(-(quad - 2.0 * cross + const))

    out_ref[...] = jnp.concatenate(logits, axis=1)


def kernel(support_features, query_features, support_labels):
    labs2d = support_labels.reshape(_N_S, 1).astype(jnp.int32)
    return pl.pallas_call(
        _proto_kernel,
        out_shape=jax.ShapeDtypeStruct((_N_Q, _C), jnp.float32),
        compiler_params=pltpu.CompilerParams(
            vmem_limit_bytes=100 * 1024 * 1024,
        ),
    )(support_features, query_features, labs2d)
